# baseline (speedup 1.0000x reference)
import functools

import jax
import jax.numpy as jnp
from jax import lax
from jax.experimental import pallas as pl
from jax.experimental.pallas import tpu as pltpu
from jax.experimental.pallas import tpu_sc as plsc

_LANES = 16


@functools.lru_cache(maxsize=None)
def _make_sc_embed(B, D, NC, NS):
    NW = NC * NS
    b_per_w = B // NW
    sizes = (32, 24, 8)
    offs = (0, 32, 56)
    slots = (0, 1, 0)
    mesh = plsc.VectorSubcoreMesh(core_axis_name="c", subcore_axis_name="s")

    @functools.partial(
        pl.kernel,
        mesh=mesh,
        out_type=jax.ShapeDtypeStruct((B, D), jnp.float32),
        scratch_types=[
            pltpu.VMEM((b_per_w,), jnp.int32),
            pltpu.VMEM((D,), jnp.int32),
            pltpu.VMEM((32, D), jnp.float32),
            pltpu.VMEM((24, D), jnp.float32),
            pltpu.SemaphoreType.DMA,
            pltpu.SemaphoreType.DMA,
            pltpu.SemaphoreType.DMA,
            pltpu.SemaphoreType.DMA,
        ],
    )
    def k(ids_hbm, table_hbm, pos_hbm, out_hbm,
          idx_v, pos_v, buf0, buf1, g0, g1, o0, o1):
        bufs = (buf0, buf1)
        gsem = (g0, g1)
        osem = (o0, o1)
        wid = lax.axis_index("s") * NC + lax.axis_index("c")
        base = wid * b_per_w
        pltpu.sync_copy(ids_hbm.at[pl.ds(base, b_per_w)], idx_v)
        pltpu.sync_copy(pos_hbm.at[0], pos_v)

        def add_pos(buf, nrows):
            def col_body(v, _):
                sl = pl.ds(v * _LANES, _LANES)
                pv = pos_v[sl].astype(jnp.float32)

                def row_body(r, _):
                    buf[r, sl] = buf[r, sl] + pv
                    return 0

                lax.fori_loop(0, nrows, row_body, 0, unroll=4)
                return 0

            lax.fori_loop(0, D // _LANES, col_body, 0)

        def start_gather(c):
            s = slots[c]
            return pltpu.async_copy(
                table_hbm.at[idx_v.at[pl.ds(offs[c], sizes[c])]],
                bufs[s].at[pl.ds(0, sizes[c])], gsem[s])

        n_chunks = len(sizes)
        gcp = [None, None]
        ocp = [None, None]
        gcp[0] = start_gather(0)
        for c in range(n_chunks):
            s = slots[c]
            gcp[s].wait()
            nxt = c + 1
            if nxt < n_chunks:
                sp = slots[nxt]
                if ocp[sp] is not None:
                    ocp[sp].wait()
                gcp[sp] = start_gather(nxt)
            add_pos(bufs[s], sizes[c])
            ocp[s] = pltpu.async_copy(
                bufs[s].at[pl.ds(0, sizes[c])],
                out_hbm.at[pl.ds(base + offs[c], sizes[c])], osem[s])
        for s in range(2):
            if ocp[s] is not None:
                ocp[s].wait()

    return k


def kernel(input_ids, embeddings, positional_id):
    B = input_ids.shape[0]
    D = embeddings.shape[1]
    info = plsc.get_sparse_core_info()
    NC, NS = info.num_cores, info.num_subcores
    if input_ids.dtype != jnp.int32:
        input_ids = input_ids.astype(jnp.int32)
    k = _make_sc_embed(B, D, NC, NS)
    return k(input_ids, embeddings, positional_id)
